# baseline (speedup 1.0000x reference)
import functools

import jax
import jax.numpy as jnp
from jax import lax
from jax.experimental import pallas as pl
from jax.experimental.pallas import tpu as pltpu
from jax.experimental.pallas import tpu_sc as plsc

EMB = 32
ROWS, COLS = 16384, 26
B = ROWS * COLS
CHUNK = 128
NGROUPS = B // CHUNK
NC, NS = 2, 16
NW = NC * NS
G_PER_W = NGROUPS // NW
NBUF = 8
NG = 4

_mesh = plsc.VectorSubcoreMesh(
    core_axis_name="c", subcore_axis_name="s", num_cores=NC, num_subcores=NS
)


@functools.partial(
    pl.kernel,
    out_type=jax.ShapeDtypeStruct((B, EMB), jnp.float32),
    mesh=_mesh,
    scratch_types=[
        pltpu.VMEM((G_PER_W, CHUNK), jnp.int32),
        pltpu.VMEM((NBUF, CHUNK, EMB), jnp.float32),
        pltpu.SemaphoreType.DMA((NBUF,)),
        pltpu.SemaphoreType.DMA((NBUF,)),
    ],
    compiler_params=pltpu.CompilerParams(use_tc_tiling_on_sc=False),
)
def _embedding_gather(idx_hbm, table_hbm, out_hbm, idx_v, bufs, gsem, wsem):
    wid = lax.axis_index("s") * NC + lax.axis_index("c")
    g0 = wid * G_PER_W
    pltpu.sync_copy(idx_hbm.at[pl.ds(g0, G_PER_W)], idx_v)

    def fire_gather(grp, slot):
        pltpu.async_copy(table_hbm.at[idx_v.at[grp]], bufs.at[slot],
                         gsem.at[slot])

    def wait_gather(grp, slot):
        pltpu.make_async_copy(table_hbm.at[idx_v.at[grp]], bufs.at[slot],
                              gsem.at[slot]).wait()

    def fire_write(grp, slot):
        pltpu.async_copy(bufs.at[slot],
                         out_hbm.at[pl.ds((g0 + grp) * CHUNK, CHUNK)],
                         wsem.at[slot])

    def wait_write(slot):
        pltpu.make_async_copy(bufs.at[slot],
                              out_hbm.at[pl.ds(g0 * CHUNK, CHUNK)],
                              wsem.at[slot]).wait()

    for b in range(NG):
        fire_gather(b, b)

    for b in range(NBUF):
        wait_gather(b, b)
        fire_write(b, b)
        nxt = b + NG
        s2 = nxt % NBUF
        if nxt >= NBUF:
            wait_write(s2)
        fire_gather(nxt, s2)

    @pl.loop(NBUF, G_PER_W - NBUF, step=NBUF)
    def _block(g):
        for b in range(NBUF):
            cur = g + b
            wait_gather(cur, b)
            fire_write(cur, b)
            s2 = (b + NG) % NBUF
            wait_write(s2)
            fire_gather(cur + NG, s2)

    for b in range(NBUF):
        cur = G_PER_W - NBUF + b
        wait_gather(cur, b)
        fire_write(cur, b)
        if b < NBUF - NG:
            s2 = (b + NG) % NBUF
            wait_write(s2)
            fire_gather(cur + NG, s2)

    for b in range(NBUF):
        wait_write(b)


def kernel(input, embed_weight):
    idx = input.reshape(NGROUPS, CHUNK)
    out = _embedding_gather(idx, embed_weight)
    return out.reshape(ROWS, COLS, EMB)
